# baseline (speedup 1.0000x reference)
import jax
import jax.numpy as jnp
import numpy as np
from jax.experimental import pallas as pl

_N = 50000
_LAMB = 1.0


def _train_idx():
    n_test = int(np.ceil(0.2 * _N))
    perm = np.random.RandomState(0).permutation(_N)
    return jnp.asarray(perm[n_test:], dtype=jnp.int32)


def _scatter_mean(src, index, dim_size, valid):
    w = valid.astype(src.dtype)
    sums = jnp.zeros((dim_size, src.shape[1]), src.dtype).at[index].add(src * w[:, None])
    cnt = jnp.zeros((dim_size,), src.dtype).at[index].add(w)
    return sums / jnp.maximum(cnt, 1.0)[:, None]


def _l2norm(x):
    n = jnp.sqrt(jnp.sum(x * x, axis=-1, keepdims=True))
    return x / jnp.maximum(n, 1e-12)


def _base_conv(x, edge_index, W, b):
    row, col = edge_index[0], edge_index[1]
    valid = row != col
    agg = _scatter_mean(x[col], row, x.shape[0], valid)
    out = jnp.concatenate([agg, x], axis=1) @ W + b
    return _l2norm(out)


def _deep_conv(x1, x2, ei_pos, ei_neg, W, b):
    n = x1.shape[0]
    loop = jnp.arange(n, dtype=ei_pos.dtype)
    ones = jnp.ones((n,), bool)
    rp, cp = ei_pos[0], ei_pos[1]
    vp = jnp.concatenate([rp != cp, ones])
    rp = jnp.concatenate([rp, loop])
    cp = jnp.concatenate([cp, loop])
    rn, cn = ei_neg[0], ei_neg[1]
    vn = jnp.concatenate([rn != cn, ones])
    rn = jnp.concatenate([rn, loop])
    cn = jnp.concatenate([cn, loop])
    out1 = _scatter_mean(x1[cp], rp, n, vp)
    out2 = _scatter_mean(x2[cn], rn, n, vn)
    out = jnp.concatenate([out1, out2, x1], axis=1) @ W + b
    return _l2norm(out)


def _passthrough(x_ref, o_ref):
    o_ref[...] = x_ref[...]


def kernel(positive_edges, negative_edges, target, pos_surrogates, neg_surrogates, X,
           W_pos_base, b_pos_base, W_neg_base, b_neg_base,
           W_pos_deep, b_pos_deep, W_neg_deep, b_neg_deep, W_reg):
    h_pos = jnp.tanh(_base_conv(X, positive_edges, W_pos_base, b_pos_base))
    h_neg = jnp.tanh(_base_conv(X, negative_edges, W_neg_base, b_neg_base))
    h_pos2 = jnp.tanh(_deep_conv(h_pos, h_neg, positive_edges, negative_edges, W_pos_deep, b_pos_deep))
    h_neg2 = jnp.tanh(_deep_conv(h_neg, h_pos, positive_edges, negative_edges, W_neg_deep, b_neg_deep))
    z = jnp.concatenate([h_pos2, h_neg2], axis=1)
    z = pl.pallas_call(
        _passthrough,
        out_shape=jax.ShapeDtypeStruct(z.shape, z.dtype),
    )(z)
    zi = z[positive_edges[0]]
    zj = z[positive_edges[1]]
    zk = z[pos_surrogates]
    n_ij = jnp.sum((zi - zj) ** 2, axis=1)
    n_ik = jnp.sum((zi - zk) ** 2, axis=1)
    l_pos = jnp.mean(jnp.maximum(n_ij - n_ik, 0.0))
    zi = z[negative_edges[0]]
    zj = z[negative_edges[1]]
    zk = z[neg_surrogates]
    n_ij = jnp.sum((zi - zj) ** 2, axis=1)
    n_ik = jnp.sum((zi - zk) ** 2, axis=1)
    l_neg = jnp.mean(jnp.maximum(n_ik - n_ij, 0.0))
    tr = _train_idx()
    preds = z[tr] @ W_reg
    logp = jax.nn.log_softmax(preds, axis=1)
    tt = target[tr]
    reg_loss = -jnp.mean(logp[jnp.arange(tt.shape[0]), tt])
    loss = reg_loss + _LAMB * (l_pos + l_neg)
    return (loss, z)


# speedup vs baseline: 4.0307x; 4.0307x over previous
import functools

import jax
import jax.numpy as jnp
import numpy as np
from jax import lax
from jax.experimental import pallas as pl
from jax.experimental.pallas import tpu as pltpu
from jax.experimental.pallas import tpu_sc as plsc

N = 50000
D = 128
E = 400000
NC = 2
NS = 16
EPT = E // NS
K = 512
NB = 50
EPT_P = NB * K
PAD = EPT_P - EPT
NP = 50048
ROWS_PT = NP // NS
ZB = ROWS_PT // K
ZT = ROWS_PT - ZB * K
ACC_ROWS = NP
DUMMY = N

RPW = 6 * E // (NC * NS)
KD = 600
NBD = RPW // KD

R = 2000
GRID_N = N // R
BE = 4000
GRID_E = E // BE

_f32 = jnp.float32
_i32 = jnp.int32

_SC_PARAMS = pltpu.CompilerParams(use_tc_tiling_on_sc=False)


def _sc_mesh():
    return plsc.VectorSubcoreMesh(
        core_axis_name="c", subcore_axis_name="s", num_cores=NC, num_subcores=NS)


def _mask_idx(ridx_v, cidx_v, sidx_c):
    def body(j, carry):
        o = j * 16
        rv = ridx_v[pl.ds(o, 16)]
        cv = cidx_v[pl.ds(o, 16)]
        sidx_c[pl.ds(o, 16)] = jnp.where(rv == cv, DUMMY, rv)
        return carry
    lax.fori_loop(0, K // 16, body, 0)


def _a1_body(er, ec, x0, x1, x2, x3, ones_hbm, zeros_hbm, scat_out, sidx_out,
             acc, cidx_dma, sidx_dma, ridx_v, cidx_v, sidx_c, rows_v, sem):
    c = lax.axis_index("c")
    s = lax.axis_index("s")

    def zero_acc():
        pltpu.sync_copy(zeros_hbm, rows_v)
        for q in range(ZB):
            pltpu.sync_copy(rows_v, acc.at[pl.ds(s * ROWS_PT + q * K, K)])
        pltpu.sync_copy(rows_v.at[pl.ds(0, ZT)],
                        acc.at[pl.ds(s * ROWS_PT + ZB * K, ZT)])

    def copy_out(plane):
        pltpu.sync_copy(acc.at[pl.ds(s * ROWS_PT, ROWS_PT)],
                        scat_out.at[plane, pl.ds(s * ROWS_PT, ROWS_PT)])

    zero_acc()
    pltpu.sync_copy(ones_hbm, rows_v)
    plsc.subcore_barrier()

    def compute_batch(b, carry):
        row = c * (NS * NB) + s * NB + b
        pltpu.sync_copy(er.at[row], ridx_v)
        pltpu.sync_copy(ec.at[row], cidx_v)
        _mask_idx(ridx_v, cidx_v, sidx_c)
        pltpu.sync_copy(sidx_c, sidx_out.at[row])
        return carry
    lax.fori_loop(0, NB, compute_batch, 0)

    def count_batch(b, carry):
        row = c * (NS * NB) + s * NB + b
        pltpu.sync_copy(sidx_out.at[row], sidx_dma)
        pltpu.sync_copy(rows_v, acc.at[sidx_dma], add=True)
        return carry
    lax.fori_loop(0, NB, count_batch, 0)

    plsc.subcore_barrier()
    copy_out(c * 5)
    plsc.subcore_barrier()

    for ch, xt in enumerate((x0, x1, x2, x3)):
        zero_acc()
        plsc.subcore_barrier()

        def chunk_batch(b, carry, xt=xt):
            row = c * (NS * NB) + s * NB + b
            pltpu.sync_copy(ec.at[row], cidx_dma)
            pltpu.sync_copy(sidx_out.at[row], sidx_dma)
            pltpu.async_copy(xt.at[cidx_dma], rows_v, sem).wait()
            pltpu.sync_copy(rows_v, acc.at[sidx_dma], add=True)
            return carry
        lax.fori_loop(0, NB, chunk_batch, 0)

        plsc.subcore_barrier()
        copy_out(c * 5 + 1 + ch)
        plsc.subcore_barrier()


_a1 = functools.partial(
    pl.kernel,
    out_type=[jax.ShapeDtypeStruct((10, NP, 32), _f32),
              jax.ShapeDtypeStruct((2 * NS * NB, K), _i32)],
    mesh=_sc_mesh(),
    compiler_params=_SC_PARAMS,
    scratch_types=[
        pltpu.VMEM_SHARED((ACC_ROWS, 32), _f32),
        pltpu.VMEM((K,), _i32),
        pltpu.VMEM((K,), _i32),
        pltpu.VMEM((K,), _i32),
        pltpu.VMEM((K,), _i32),
        pltpu.VMEM((K,), _i32),
        pltpu.VMEM((K, 32), _f32),
        pltpu.SemaphoreType.DMA,
    ],
)(_a1_body)


def _a2_body(ec, sidx_in, hp, hn, zeros_hbm, scat_out,
             acc, cidx_dma, sidx_dma, rows_v, sem):
    c = lax.axis_index("c")
    s = lax.axis_index("s")

    def zero_acc():
        pltpu.sync_copy(zeros_hbm, rows_v)
        for q in range(ZB):
            pltpu.sync_copy(rows_v, acc.at[pl.ds(s * ROWS_PT + q * K, K)])
        pltpu.sync_copy(rows_v.at[pl.ds(0, ZT)],
                        acc.at[pl.ds(s * ROWS_PT + ZB * K, ZT)])

    for t, tab in enumerate((hp, hn)):
        zero_acc()
        plsc.subcore_barrier()

        def batch(b, carry, tab=tab):
            row = c * (NS * NB) + s * NB + b
            pltpu.sync_copy(ec.at[row], cidx_dma)
            pltpu.sync_copy(sidx_in.at[row], sidx_dma)
            pltpu.async_copy(tab.at[cidx_dma], rows_v, sem).wait()
            pltpu.sync_copy(rows_v, acc.at[sidx_dma], add=True)
            return carry
        lax.fori_loop(0, NB, batch, 0)

        plsc.subcore_barrier()
        pltpu.sync_copy(acc.at[pl.ds(s * ROWS_PT, ROWS_PT)],
                        scat_out.at[c * 2 + t, pl.ds(s * ROWS_PT, ROWS_PT)])
        plsc.subcore_barrier()


_a2 = functools.partial(
    pl.kernel,
    out_type=jax.ShapeDtypeStruct((4, NP, 32), _f32),
    mesh=_sc_mesh(),
    compiler_params=_SC_PARAMS,
    scratch_types=[
        pltpu.VMEM_SHARED((ACC_ROWS, 32), _f32),
        pltpu.VMEM((K,), _i32),
        pltpu.VMEM((K,), _i32),
        pltpu.VMEM((K, 32), _f32),
        pltpu.SemaphoreType.DMA,
    ],
)(_a2_body)


def _d_body(z_hbm, idx_hbm, out_hbm, idxb, rows, sem):
    c = lax.axis_index("c")
    s = lax.axis_index("s")
    w = s * NC + c
    r0 = w * RPW

    def batch(b, carry):
        base = r0 + b * KD
        pltpu.sync_copy(idx_hbm.at[w * NBD + b], idxb)
        pltpu.async_copy(z_hbm.at[idxb], rows, sem).wait()
        pltpu.sync_copy(rows, out_hbm.at[pl.ds(base, KD)])
        return carry
    lax.fori_loop(0, NBD, batch, 0)


_d = functools.partial(
    pl.kernel,
    out_type=jax.ShapeDtypeStruct((6 * E, 64), _f32),
    mesh=_sc_mesh(),
    compiler_params=_SC_PARAMS,
    scratch_types=[
        pltpu.VMEM((KD,), _i32),
        pltpu.VMEM((KD, 64), _f32),
        pltpu.SemaphoreType.DMA,
    ],
)(_d_body)


def _l2t(acc):
    nrm = jnp.sqrt(jnp.sum(acc * acc, axis=1, keepdims=True))
    return jnp.tanh(acc / jnp.maximum(nrm, 1e-12))


def _b_body(x_ref, sc_ref, wp_ref, bp_ref, wn_ref, bn_ref, hp_ref, hn_ref):
    x = x_ref[...]

    def one(w, b, off):
        cnt = jnp.maximum(sc_ref[off, :, 0:1], 1.0)
        acc = jnp.dot(x, w[128:256], preferred_element_type=_f32) + b
        for ch in range(4):
            acc = acc + jnp.dot(sc_ref[off + 1 + ch] / cnt,
                                w[32 * ch:32 * ch + 32],
                                preferred_element_type=_f32)
        return _l2t(acc)

    hp_ref[...] = one(wp_ref[...], bp_ref[...], 0)
    hn_ref[...] = one(wn_ref[...], bn_ref[...], 5)


def _b_call(X, scat, wp, bp, wn, bn):
    return pl.pallas_call(
        _b_body,
        grid=(GRID_N,),
        in_specs=[
            pl.BlockSpec((R, D), lambda i: (i, 0)),
            pl.BlockSpec((10, R, 32), lambda i: (0, i, 0)),
            pl.BlockSpec((2 * D, 32), lambda i: (0, 0)),
            pl.BlockSpec((1, 32), lambda i: (0, 0)),
            pl.BlockSpec((2 * D, 32), lambda i: (0, 0)),
            pl.BlockSpec((1, 32), lambda i: (0, 0)),
        ],
        out_specs=[
            pl.BlockSpec((R, 32), lambda i: (i, 0)),
            pl.BlockSpec((R, 32), lambda i: (i, 0)),
        ],
        out_shape=[jax.ShapeDtypeStruct((N, 32), _f32),
                   jax.ShapeDtypeStruct((N, 32), _f32)],
    )(X, scat, wp, bp.reshape(1, 32), wn, bn.reshape(1, 32))


def _c_body(d4_ref, cp_ref, cn_ref, hp_ref, hn_ref, wpd_ref, bpd_ref,
            wnd_ref, bnd_ref, wreg_ref, targ_ref, mask_ref, z_ref, ls_ref):
    i = pl.program_id(0)
    hp = hp_ref[...]
    hn = hn_ref[...]
    cp1 = cp_ref[0, :, 0:1] + 1.0
    cn1 = cn_ref[0, :, 0:1] + 1.0

    def deep(o1, o2, x1, w, b):
        acc = (jnp.dot(o1, w[0:32], preferred_element_type=_f32)
               + jnp.dot(o2, w[32:64], preferred_element_type=_f32)
               + jnp.dot(x1, w[64:96], preferred_element_type=_f32) + b)
        return _l2t(acc)

    a4 = d4_ref[0]
    b4 = d4_ref[1]
    c4 = d4_ref[2]
    d4 = d4_ref[3]
    zp = deep((a4 + hp) / cp1, (d4 + hn) / cn1, hp, wpd_ref[...], bpd_ref[...])
    zn = deep((b4 + hn) / cp1, (c4 + hp) / cn1, hn, wnd_ref[...], bnd_ref[...])
    z = jnp.concatenate([zp, zn], axis=1)
    z_ref[...] = z

    w = wreg_ref[...]
    p0 = jnp.dot(z, w[:, 0:1], preferred_element_type=_f32)
    p1 = jnp.dot(z, w[:, 1:2], preferred_element_type=_f32)
    m = jnp.maximum(p0, p1)
    lse = m + jnp.log(jnp.exp(p0 - m) + jnp.exp(p1 - m))
    tt = targ_ref[...]
    picked = jnp.where(tt == 0, p0, p1)
    contrib = mask_ref[...] * (lse - picked)

    @pl.when(i == 0)
    def _():
        ls_ref[...] = jnp.zeros((1, 1), _f32)
    ls_ref[...] += jnp.sum(contrib).reshape(1, 1)


def _c_call(d4, scat, hp, hn, wpd, bpd, wnd, bnd, wreg, targ2d, mask2d):
    return pl.pallas_call(
        _c_body,
        grid=(GRID_N,),
        in_specs=[
            pl.BlockSpec((4, R, 32), lambda i: (0, i, 0)),
            pl.BlockSpec((1, R, 32), lambda i: (0, i, 0)),
            pl.BlockSpec((1, R, 32), lambda i: (5, i, 0)),
            pl.BlockSpec((R, 32), lambda i: (i, 0)),
            pl.BlockSpec((R, 32), lambda i: (i, 0)),
            pl.BlockSpec((96, 32), lambda i: (0, 0)),
            pl.BlockSpec((1, 32), lambda i: (0, 0)),
            pl.BlockSpec((96, 32), lambda i: (0, 0)),
            pl.BlockSpec((1, 32), lambda i: (0, 0)),
            pl.BlockSpec((64, 2), lambda i: (0, 0)),
            pl.BlockSpec((R, 1), lambda i: (i, 0)),
            pl.BlockSpec((R, 1), lambda i: (i, 0)),
        ],
        out_specs=[
            pl.BlockSpec((R, 64), lambda i: (i, 0)),
            pl.BlockSpec((1, 1), lambda i: (0, 0)),
        ],
        out_shape=[jax.ShapeDtypeStruct((N, 64), _f32),
                   jax.ShapeDtypeStruct((1, 1), _f32)],
    )(d4, scat, scat, hp, hn, wpd, bpd.reshape(1, 32), wnd, bnd.reshape(1, 32),
      wreg, targ2d, mask2d)


def _e_body(zi_p, zj_p, zk_p, zi_n, zj_n, zk_n, lp_ref, ln_ref):
    i = pl.program_id(0)

    def dist(a, b):
        d = a[0] - b[0]
        return jnp.sum(d * d, axis=1)

    nij = dist(zi_p, zj_p)
    nik = dist(zi_p, zk_p)
    lp = jnp.sum(jnp.maximum(nij - nik, 0.0))
    nij2 = dist(zi_n, zj_n)
    nik2 = dist(zi_n, zk_n)
    ln = jnp.sum(jnp.maximum(nik2 - nij2, 0.0))

    @pl.when(i == 0)
    def _():
        lp_ref[...] = jnp.zeros((1, 1), _f32)
        ln_ref[...] = jnp.zeros((1, 1), _f32)
    lp_ref[...] += lp.reshape(1, 1)
    ln_ref[...] += ln.reshape(1, 1)


def _e_call(gz6):
    spec = lambda j: pl.BlockSpec((1, BE, 64), lambda i, j=j: (j, i, 0))
    return pl.pallas_call(
        _e_body,
        grid=(GRID_E,),
        in_specs=[spec(0), spec(1), spec(2), spec(3), spec(4), spec(5)],
        out_specs=[pl.BlockSpec((1, 1), lambda i: (0, 0)),
                   pl.BlockSpec((1, 1), lambda i: (0, 0))],
        out_shape=[jax.ShapeDtypeStruct((1, 1), _f32),
                   jax.ShapeDtypeStruct((1, 1), _f32)],
    )(gz6, gz6, gz6, gz6, gz6, gz6)


def _host_train_mask():
    n_test = int(np.ceil(0.2 * N))
    perm = np.random.RandomState(0).permutation(N)
    m = np.zeros((N, 1), np.float32)
    m[perm[n_test:]] = 1.0
    return m


_MASK = _host_train_mask()
_N_TRAIN = int(_MASK.sum())


def kernel(positive_edges, negative_edges, target, pos_surrogates, neg_surrogates, X,
           W_pos_base, b_pos_base, W_neg_base, b_neg_base,
           W_pos_deep, b_pos_deep, W_neg_deep, b_neg_deep, W_reg):
    rp, cp = positive_edges[0], positive_edges[1]
    rn, cn = negative_edges[0], negative_edges[1]

    def padded(v, fill):
        return jnp.pad(v.reshape(NS, EPT), ((0, 0), (0, PAD)),
                       constant_values=fill)

    er = jnp.concatenate([padded(rp, DUMMY), padded(rn, DUMMY)]
                         ).reshape(2 * NS * NB, K)
    ec = jnp.concatenate([padded(cp, 0), padded(cn, 0)]
                         ).reshape(2 * NS * NB, K)
    xs = X.reshape(N, 4, 32).transpose(1, 0, 2)

    scat, sidx = _a1(er, ec, xs[0], xs[1], xs[2], xs[3],
                     jnp.ones((K, 32), _f32), jnp.zeros((K, 32), _f32))
    h_pos, h_neg = _b_call(X, scat, W_pos_base, b_pos_base, W_neg_base, b_neg_base)
    d4 = _a2(ec, sidx, h_pos, h_neg, jnp.zeros((K, 32), _f32))
    z, reg_sum = _c_call(d4, scat, h_pos, h_neg, W_pos_deep, b_pos_deep,
                         W_neg_deep, b_neg_deep, W_reg,
                         target.reshape(N, 1),
                         jnp.asarray(_MASK))
    idx_all = jnp.concatenate(
        [rp, cp, pos_surrogates, rn, cn, neg_surrogates]).reshape(6 * E // KD, KD)
    gz = _d(z, idx_all)
    lp_sum, ln_sum = _e_call(gz.reshape(6, E, 64))
    loss = (reg_sum[0, 0] / _N_TRAIN
            + (lp_sum[0, 0] + ln_sum[0, 0]) / float(E))
    return (loss, z)


# speedup vs baseline: 4.2731x; 1.0601x over previous
import functools

import jax
import jax.numpy as jnp
import numpy as np
from jax import lax
from jax.experimental import pallas as pl
from jax.experimental.pallas import tpu as pltpu
from jax.experimental.pallas import tpu_sc as plsc

N = 50000
D = 128
E = 400000
NC = 2
NS = 16
EPT = E // NS
K = 320
NB = 80
EPT_P = NB * K
PAD = EPT_P - EPT
NP = 50048
ROWS_PT = NP // NS
ZB = ROWS_PT // K
ZT = ROWS_PT - ZB * K
ACC_ROWS = NP
DUMMY = N

RPW = 6 * E // (NC * NS)
KD = 600
NBD = RPW // KD

R = 2000
GRID_N = N // R
BE = 4000
GRID_E = E // BE

_f32 = jnp.float32
_i32 = jnp.int32

_SC_PARAMS = pltpu.CompilerParams(use_tc_tiling_on_sc=False)


def _sc_mesh():
    return plsc.VectorSubcoreMesh(
        core_axis_name="c", subcore_axis_name="s", num_cores=NC, num_subcores=NS)


def _mask_idx(ridx_v, cidx_v, sidx_c):
    def body(j, carry):
        o = j * 16
        rv = ridx_v[pl.ds(o, 16)]
        cv = cidx_v[pl.ds(o, 16)]
        sidx_c[pl.ds(o, 16)] = jnp.where(rv == cv, DUMMY, rv)
        return carry
    lax.fori_loop(0, K // 16, body, 0)


def _a1_body(er, ec, x0, x1, x2, x3, ones_hbm, zeros_hbm, scat_out, sidx_out,
             acc, cidx0, cidx1, sidx0, sidx1, ridx_v, cidx_v, sidx_c,
             rows0, rows1, sem0, sem1):
    c = lax.axis_index("c")
    s = lax.axis_index("s")

    def zero_acc():
        pltpu.sync_copy(zeros_hbm, rows0)
        for q in range(ZB):
            pltpu.sync_copy(rows0, acc.at[pl.ds(s * ROWS_PT + q * K, K)])
        pltpu.sync_copy(rows0.at[pl.ds(0, ZT)],
                        acc.at[pl.ds(s * ROWS_PT + ZB * K, ZT)])

    def copy_out(plane):
        pltpu.sync_copy(acc.at[pl.ds(s * ROWS_PT, ROWS_PT)],
                        scat_out.at[plane, pl.ds(s * ROWS_PT, ROWS_PT)])

    def rowof(b):
        return c * (NS * NB) + s * NB + b

    def compute_batch(b, carry):
        row = rowof(b)
        pltpu.sync_copy(er.at[row], ridx_v)
        pltpu.sync_copy(ec.at[row], cidx_v)
        _mask_idx(ridx_v, cidx_v, sidx_c)
        pltpu.sync_copy(sidx_c, sidx_out.at[row])
        return carry
    lax.fori_loop(0, NB, compute_batch, 0)

    zero_acc()
    pltpu.sync_copy(ones_hbm, rows0)
    plsc.subcore_barrier()

    def count_batch(b, carry):
        pltpu.sync_copy(sidx_out.at[rowof(b)], sidx0)
        pltpu.sync_copy(rows0, acc.at[sidx0], add=True)
        return carry
    lax.fori_loop(0, NB, count_batch, 0)

    plsc.subcore_barrier()
    copy_out(c * 5)
    plsc.subcore_barrier()

    for ch, xt in enumerate((x0, x1, x2, x3)):
        zero_acc()
        plsc.subcore_barrier()

        pltpu.sync_copy(ec.at[rowof(0)], cidx0)
        pltpu.sync_copy(sidx_out.at[rowof(0)], sidx0)
        pltpu.async_copy(xt.at[cidx0], rows0, sem0)

        def pipe(i, carry, xt=xt):
            b0 = 2 * i
            pltpu.sync_copy(ec.at[rowof(b0 + 1)], cidx1)
            pltpu.sync_copy(sidx_out.at[rowof(b0 + 1)], sidx1)
            pltpu.async_copy(xt.at[cidx1], rows1, sem1)
            pltpu.make_async_copy(xt.at[cidx0], rows0, sem0).wait()
            pltpu.sync_copy(rows0, acc.at[sidx0], add=True)
            pltpu.sync_copy(ec.at[rowof(b0 + 2)], cidx0)
            pltpu.sync_copy(sidx_out.at[rowof(b0 + 2)], sidx0)
            pltpu.async_copy(xt.at[cidx0], rows0, sem0)
            pltpu.make_async_copy(xt.at[cidx1], rows1, sem1).wait()
            pltpu.sync_copy(rows1, acc.at[sidx1], add=True)
            return carry
        lax.fori_loop(0, NB // 2 - 1, pipe, 0)

        pltpu.sync_copy(ec.at[rowof(NB - 1)], cidx1)
        pltpu.sync_copy(sidx_out.at[rowof(NB - 1)], sidx1)
        pltpu.async_copy(xt.at[cidx1], rows1, sem1)
        pltpu.make_async_copy(xt.at[cidx0], rows0, sem0).wait()
        pltpu.sync_copy(rows0, acc.at[sidx0], add=True)
        pltpu.make_async_copy(xt.at[cidx1], rows1, sem1).wait()
        pltpu.sync_copy(rows1, acc.at[sidx1], add=True)

        plsc.subcore_barrier()
        copy_out(c * 5 + 1 + ch)
        plsc.subcore_barrier()


_a1 = functools.partial(
    pl.kernel,
    out_type=[jax.ShapeDtypeStruct((10, NP, 32), _f32),
              jax.ShapeDtypeStruct((2 * NS * NB, K), _i32)],
    mesh=_sc_mesh(),
    compiler_params=_SC_PARAMS,
    scratch_types=[
        pltpu.VMEM_SHARED((ACC_ROWS, 32), _f32),
        pltpu.VMEM((K,), _i32),
        pltpu.VMEM((K,), _i32),
        pltpu.VMEM((K,), _i32),
        pltpu.VMEM((K,), _i32),
        pltpu.VMEM((K,), _i32),
        pltpu.VMEM((K,), _i32),
        pltpu.VMEM((K,), _i32),
        pltpu.VMEM((K, 32), _f32),
        pltpu.VMEM((K, 32), _f32),
        pltpu.SemaphoreType.DMA,
        pltpu.SemaphoreType.DMA,
    ],
)(_a1_body)


def _a2_body(ec, sidx_in, hp, hn, zeros_hbm, scat_out,
             acc, cidx0, cidx1, sidx0, sidx1, rows0, rows1, sem0, sem1):
    c = lax.axis_index("c")
    s = lax.axis_index("s")

    def zero_acc():
        pltpu.sync_copy(zeros_hbm, rows0)
        for q in range(ZB):
            pltpu.sync_copy(rows0, acc.at[pl.ds(s * ROWS_PT + q * K, K)])
        pltpu.sync_copy(rows0.at[pl.ds(0, ZT)],
                        acc.at[pl.ds(s * ROWS_PT + ZB * K, ZT)])

    def rowof(b):
        return c * (NS * NB) + s * NB + b

    for t, tab in enumerate((hp, hn)):
        zero_acc()
        plsc.subcore_barrier()

        pltpu.sync_copy(ec.at[rowof(0)], cidx0)
        pltpu.sync_copy(sidx_in.at[rowof(0)], sidx0)
        pltpu.async_copy(tab.at[cidx0], rows0, sem0)

        def pipe(i, carry, tab=tab):
            b0 = 2 * i
            pltpu.sync_copy(ec.at[rowof(b0 + 1)], cidx1)
            pltpu.sync_copy(sidx_in.at[rowof(b0 + 1)], sidx1)
            pltpu.async_copy(tab.at[cidx1], rows1, sem1)
            pltpu.make_async_copy(tab.at[cidx0], rows0, sem0).wait()
            pltpu.sync_copy(rows0, acc.at[sidx0], add=True)
            pltpu.sync_copy(ec.at[rowof(b0 + 2)], cidx0)
            pltpu.sync_copy(sidx_in.at[rowof(b0 + 2)], sidx0)
            pltpu.async_copy(tab.at[cidx0], rows0, sem0)
            pltpu.make_async_copy(tab.at[cidx1], rows1, sem1).wait()
            pltpu.sync_copy(rows1, acc.at[sidx1], add=True)
            return carry
        lax.fori_loop(0, NB // 2 - 1, pipe, 0)

        pltpu.sync_copy(ec.at[rowof(NB - 1)], cidx1)
        pltpu.sync_copy(sidx_in.at[rowof(NB - 1)], sidx1)
        pltpu.async_copy(tab.at[cidx1], rows1, sem1)
        pltpu.make_async_copy(tab.at[cidx0], rows0, sem0).wait()
        pltpu.sync_copy(rows0, acc.at[sidx0], add=True)
        pltpu.make_async_copy(tab.at[cidx1], rows1, sem1).wait()
        pltpu.sync_copy(rows1, acc.at[sidx1], add=True)

        plsc.subcore_barrier()
        pltpu.sync_copy(acc.at[pl.ds(s * ROWS_PT, ROWS_PT)],
                        scat_out.at[c * 2 + t, pl.ds(s * ROWS_PT, ROWS_PT)])
        plsc.subcore_barrier()


_a2 = functools.partial(
    pl.kernel,
    out_type=jax.ShapeDtypeStruct((4, NP, 32), _f32),
    mesh=_sc_mesh(),
    compiler_params=_SC_PARAMS,
    scratch_types=[
        pltpu.VMEM_SHARED((ACC_ROWS, 32), _f32),
        pltpu.VMEM((K,), _i32),
        pltpu.VMEM((K,), _i32),
        pltpu.VMEM((K,), _i32),
        pltpu.VMEM((K,), _i32),
        pltpu.VMEM((K, 32), _f32),
        pltpu.VMEM((K, 32), _f32),
        pltpu.SemaphoreType.DMA,
        pltpu.SemaphoreType.DMA,
    ],
)(_a2_body)


def _d_body(z_hbm, idx_hbm, out_hbm, idx0, idx1, rows0, rows1, sem0, sem1):
    c = lax.axis_index("c")
    s = lax.axis_index("s")
    w = s * NC + c
    r0 = w * RPW

    pltpu.sync_copy(idx_hbm.at[w * NBD], idx0)
    pltpu.async_copy(z_hbm.at[idx0], rows0, sem0)

    def pipe(i, carry):
        b0 = 2 * i
        pltpu.sync_copy(idx_hbm.at[w * NBD + b0 + 1], idx1)
        pltpu.async_copy(z_hbm.at[idx1], rows1, sem1)
        pltpu.make_async_copy(z_hbm.at[idx0], rows0, sem0).wait()
        pltpu.sync_copy(rows0, out_hbm.at[pl.ds(r0 + b0 * KD, KD)])
        pltpu.sync_copy(idx_hbm.at[w * NBD + b0 + 2], idx0)
        pltpu.async_copy(z_hbm.at[idx0], rows0, sem0)
        pltpu.make_async_copy(z_hbm.at[idx1], rows1, sem1).wait()
        pltpu.sync_copy(rows1, out_hbm.at[pl.ds(r0 + (b0 + 1) * KD, KD)])
        return carry
    lax.fori_loop(0, (NBD - 1) // 2, pipe, 0)

    pltpu.make_async_copy(z_hbm.at[idx0], rows0, sem0).wait()
    pltpu.sync_copy(rows0, out_hbm.at[pl.ds(r0 + (NBD - 1) * KD, KD)])


_d = functools.partial(
    pl.kernel,
    out_type=jax.ShapeDtypeStruct((6 * E, 64), _f32),
    mesh=_sc_mesh(),
    compiler_params=_SC_PARAMS,
    scratch_types=[
        pltpu.VMEM((KD,), _i32),
        pltpu.VMEM((KD,), _i32),
        pltpu.VMEM((KD, 64), _f32),
        pltpu.VMEM((KD, 64), _f32),
        pltpu.SemaphoreType.DMA,
        pltpu.SemaphoreType.DMA,
    ],
)(_d_body)


def _l2t(acc):
    nrm = jnp.sqrt(jnp.sum(acc * acc, axis=1, keepdims=True))
    return jnp.tanh(acc / jnp.maximum(nrm, 1e-12))


def _b_body(x_ref, sc_ref, wp_ref, bp_ref, wn_ref, bn_ref, hp_ref, hn_ref):
    x = x_ref[...]

    def one(w, b, off):
        cnt = jnp.maximum(sc_ref[off, :, 0:1], 1.0)
        acc = jnp.dot(x, w[128:256], preferred_element_type=_f32) + b
        for ch in range(4):
            acc = acc + jnp.dot(sc_ref[off + 1 + ch] / cnt,
                                w[32 * ch:32 * ch + 32],
                                preferred_element_type=_f32)
        return _l2t(acc)

    hp_ref[...] = one(wp_ref[...], bp_ref[...], 0)
    hn_ref[...] = one(wn_ref[...], bn_ref[...], 5)


def _b_call(X, scat, wp, bp, wn, bn):
    return pl.pallas_call(
        _b_body,
        grid=(GRID_N,),
        in_specs=[
            pl.BlockSpec((R, D), lambda i: (i, 0)),
            pl.BlockSpec((10, R, 32), lambda i: (0, i, 0)),
            pl.BlockSpec((2 * D, 32), lambda i: (0, 0)),
            pl.BlockSpec((1, 32), lambda i: (0, 0)),
            pl.BlockSpec((2 * D, 32), lambda i: (0, 0)),
            pl.BlockSpec((1, 32), lambda i: (0, 0)),
        ],
        out_specs=[
            pl.BlockSpec((R, 32), lambda i: (i, 0)),
            pl.BlockSpec((R, 32), lambda i: (i, 0)),
        ],
        out_shape=[jax.ShapeDtypeStruct((N, 32), _f32),
                   jax.ShapeDtypeStruct((N, 32), _f32)],
    )(X, scat, wp, bp.reshape(1, 32), wn, bn.reshape(1, 32))


def _c_body(d4_ref, cp_ref, cn_ref, hp_ref, hn_ref, wpd_ref, bpd_ref,
            wnd_ref, bnd_ref, wreg_ref, targ_ref, mask_ref, z_ref, ls_ref):
    i = pl.program_id(0)
    hp = hp_ref[...]
    hn = hn_ref[...]
    cp1 = cp_ref[0, :, 0:1] + 1.0
    cn1 = cn_ref[0, :, 0:1] + 1.0

    def deep(o1, o2, x1, w, b):
        acc = (jnp.dot(o1, w[0:32], preferred_element_type=_f32)
               + jnp.dot(o2, w[32:64], preferred_element_type=_f32)
               + jnp.dot(x1, w[64:96], preferred_element_type=_f32) + b)
        return _l2t(acc)

    a4 = d4_ref[0]
    b4 = d4_ref[1]
    c4 = d4_ref[2]
    d4 = d4_ref[3]
    zp = deep((a4 + hp) / cp1, (d4 + hn) / cn1, hp, wpd_ref[...], bpd_ref[...])
    zn = deep((b4 + hn) / cp1, (c4 + hp) / cn1, hn, wnd_ref[...], bnd_ref[...])
    z = jnp.concatenate([zp, zn], axis=1)
    z_ref[...] = z

    w = wreg_ref[...]
    p0 = jnp.dot(z, w[:, 0:1], preferred_element_type=_f32)
    p1 = jnp.dot(z, w[:, 1:2], preferred_element_type=_f32)
    m = jnp.maximum(p0, p1)
    lse = m + jnp.log(jnp.exp(p0 - m) + jnp.exp(p1 - m))
    tt = targ_ref[...]
    picked = jnp.where(tt == 0, p0, p1)
    contrib = mask_ref[...] * (lse - picked)

    @pl.when(i == 0)
    def _():
        ls_ref[...] = jnp.zeros((1, 1), _f32)
    ls_ref[...] += jnp.sum(contrib).reshape(1, 1)


def _c_call(d4, scat, hp, hn, wpd, bpd, wnd, bnd, wreg, targ2d, mask2d):
    return pl.pallas_call(
        _c_body,
        grid=(GRID_N,),
        in_specs=[
            pl.BlockSpec((4, R, 32), lambda i: (0, i, 0)),
            pl.BlockSpec((1, R, 32), lambda i: (0, i, 0)),
            pl.BlockSpec((1, R, 32), lambda i: (5, i, 0)),
            pl.BlockSpec((R, 32), lambda i: (i, 0)),
            pl.BlockSpec((R, 32), lambda i: (i, 0)),
            pl.BlockSpec((96, 32), lambda i: (0, 0)),
            pl.BlockSpec((1, 32), lambda i: (0, 0)),
            pl.BlockSpec((96, 32), lambda i: (0, 0)),
            pl.BlockSpec((1, 32), lambda i: (0, 0)),
            pl.BlockSpec((64, 2), lambda i: (0, 0)),
            pl.BlockSpec((R, 1), lambda i: (i, 0)),
            pl.BlockSpec((R, 1), lambda i: (i, 0)),
        ],
        out_specs=[
            pl.BlockSpec((R, 64), lambda i: (i, 0)),
            pl.BlockSpec((1, 1), lambda i: (0, 0)),
        ],
        out_shape=[jax.ShapeDtypeStruct((N, 64), _f32),
                   jax.ShapeDtypeStruct((1, 1), _f32)],
    )(d4, scat, scat, hp, hn, wpd, bpd.reshape(1, 32), wnd, bnd.reshape(1, 32),
      wreg, targ2d, mask2d)


def _e_body(zi_p, zj_p, zk_p, zi_n, zj_n, zk_n, lp_ref, ln_ref):
    i = pl.program_id(0)

    def dist(a, b):
        d = a[0] - b[0]
        return jnp.sum(d * d, axis=1)

    nij = dist(zi_p, zj_p)
    nik = dist(zi_p, zk_p)
    lp = jnp.sum(jnp.maximum(nij - nik, 0.0))
    nij2 = dist(zi_n, zj_n)
    nik2 = dist(zi_n, zk_n)
    ln = jnp.sum(jnp.maximum(nik2 - nij2, 0.0))

    @pl.when(i == 0)
    def _():
        lp_ref[...] = jnp.zeros((1, 1), _f32)
        ln_ref[...] = jnp.zeros((1, 1), _f32)
    lp_ref[...] += lp.reshape(1, 1)
    ln_ref[...] += ln.reshape(1, 1)


def _e_call(gz6):
    spec = lambda j: pl.BlockSpec((1, BE, 64), lambda i, j=j: (j, i, 0))
    return pl.pallas_call(
        _e_body,
        grid=(GRID_E,),
        in_specs=[spec(0), spec(1), spec(2), spec(3), spec(4), spec(5)],
        out_specs=[pl.BlockSpec((1, 1), lambda i: (0, 0)),
                   pl.BlockSpec((1, 1), lambda i: (0, 0))],
        out_shape=[jax.ShapeDtypeStruct((1, 1), _f32),
                   jax.ShapeDtypeStruct((1, 1), _f32)],
    )(gz6, gz6, gz6, gz6, gz6, gz6)


def _host_train_mask():
    n_test = int(np.ceil(0.2 * N))
    perm = np.random.RandomState(0).permutation(N)
    m = np.zeros((N, 1), np.float32)
    m[perm[n_test:]] = 1.0
    return m


_MASK = _host_train_mask()
_N_TRAIN = int(_MASK.sum())


def kernel(positive_edges, negative_edges, target, pos_surrogates, neg_surrogates, X,
           W_pos_base, b_pos_base, W_neg_base, b_neg_base,
           W_pos_deep, b_pos_deep, W_neg_deep, b_neg_deep, W_reg):
    rp, cp = positive_edges[0], positive_edges[1]
    rn, cn = negative_edges[0], negative_edges[1]

    def padded(v, fill):
        return jnp.pad(v.reshape(NS, EPT), ((0, 0), (0, PAD)),
                       constant_values=fill)

    er = jnp.concatenate([padded(rp, DUMMY), padded(rn, DUMMY)]
                         ).reshape(2 * NS * NB, K)
    ec = jnp.concatenate([padded(cp, 0), padded(cn, 0)]
                         ).reshape(2 * NS * NB, K)
    xs = X.reshape(N, 4, 32).transpose(1, 0, 2)

    scat, sidx = _a1(er, ec, xs[0], xs[1], xs[2], xs[3],
                     jnp.ones((K, 32), _f32), jnp.zeros((K, 32), _f32))
    h_pos, h_neg = _b_call(X, scat, W_pos_base, b_pos_base, W_neg_base, b_neg_base)
    d4 = _a2(ec, sidx, h_pos, h_neg, jnp.zeros((K, 32), _f32))
    z, reg_sum = _c_call(d4, scat, h_pos, h_neg, W_pos_deep, b_pos_deep,
                         W_neg_deep, b_neg_deep, W_reg,
                         target.reshape(N, 1),
                         jnp.asarray(_MASK))
    idx_all = jnp.concatenate(
        [rp, cp, pos_surrogates, rn, cn, neg_surrogates]).reshape(6 * E // KD, KD)
    gz = _d(z, idx_all)
    lp_sum, ln_sum = _e_call(gz.reshape(6, E, 64))
    loss = (reg_sum[0, 0] / _N_TRAIN
            + (lp_sum[0, 0] + ln_sum[0, 0]) / float(E))
    return (loss, z)
